# Initial kernel scaffold; baseline (speedup 1.0000x reference)
import jax
import jax.numpy as jnp
from jax.experimental import pallas as pl


def kernel(user_embed, item_embed, ui_rows, ui_cols, WQ, bQ, WK, bK):
    raise NotImplementedError("write your pallas kernel here")


# speedup vs baseline: 41.4308x; 41.4308x over previous
import functools

import jax
import jax.numpy as jnp
from jax import lax
from jax.experimental import pallas as pl
from jax.experimental.pallas import tpu as pltpu
from jax.experimental.pallas import tpu_sc as plsc

N_USERS = 4096
N_ITEMS = 8192
NNZ = 262144
EMBED = 128
ATT = 128
TAU = 0.5
INV_TAU = 2.0

NSUB = 16
CHUNK = NNZ // NSUB
GROWS = 128
GCOLS = 128
SEGS = N_ITEMS
SLICE = SEGS // NSUB


def _proj_body(x_ref, wq_ref, bq_ref, wk_ref, bk_ref, q_ref, k_ref):
    x = x_ref[...]
    q_ref[...] = (
        jnp.dot(x, wq_ref[...], preferred_element_type=jnp.float32) + bq_ref[...]
    )
    k_ref[...] = (
        jnp.dot(x, wk_ref[...], preferred_element_type=jnp.float32) + bk_ref[...]
    )


def _projections(X, WQ, bQ, WK, bK):
    n = X.shape[0]
    blk = 1024
    grid = (n // blk,)
    return pl.pallas_call(
        _proj_body,
        grid=grid,
        in_specs=[
            pl.BlockSpec((blk, EMBED), lambda i: (i, 0)),
            pl.BlockSpec((EMBED, ATT), lambda i: (0, 0)),
            pl.BlockSpec((1, ATT), lambda i: (0, 0)),
            pl.BlockSpec((EMBED, ATT), lambda i: (0, 0)),
            pl.BlockSpec((1, ATT), lambda i: (0, 0)),
        ],
        out_specs=[
            pl.BlockSpec((blk, ATT), lambda i: (i, 0)),
            pl.BlockSpec((blk, ATT), lambda i: (i, 0)),
        ],
        out_shape=[
            jax.ShapeDtypeStruct((n, ATT), jnp.float32),
            jax.ShapeDtypeStruct((n, ATT), jnp.float32),
        ],
    )(X, WQ, bQ.reshape(1, ATT), WK, bK.reshape(1, ATT))


_SLAB = 128
_NSLAB = N_ITEMS // _SLAB


def _dense_body(qu_ref, ku_ref, qi_ref, ki_ref, m1_ref, m2_ref):
    dn = (((1,), (1,)), ((), ()))
    m1_ref[0] = lax.dot_general(
        qu_ref[...], ki_ref[...], dn, preferred_element_type=jnp.float32
    )
    m2_ref[0] = lax.dot_general(
        ku_ref[...], qi_ref[...], dn, preferred_element_type=jnp.float32
    )


def _dense_logits(QX, KX):
    grid = (_NSLAB,)
    ioff = N_USERS // _SLAB
    return pl.pallas_call(
        _dense_body,
        grid=grid,
        in_specs=[
            pl.BlockSpec((N_USERS, ATT), lambda c: (0, 0)),
            pl.BlockSpec((N_USERS, ATT), lambda c: (0, 0)),
            pl.BlockSpec((_SLAB, ATT), lambda c: (c + ioff, 0)),
            pl.BlockSpec((_SLAB, ATT), lambda c: (c + ioff, 0)),
        ],
        out_specs=[
            pl.BlockSpec((1, N_USERS, _SLAB), lambda c: (c, 0, 0)),
            pl.BlockSpec((1, N_USERS, _SLAB), lambda c: (c, 0, 0)),
        ],
        out_shape=[
            jax.ShapeDtypeStruct((_NSLAB, N_USERS, _SLAB), jnp.float32),
            jax.ShapeDtypeStruct((_NSLAB, N_USERS, _SLAB), jnp.float32),
        ],
    )(QX, KX, QX, KX)


def _sc_body(
    m1_ref, m2_ref, rows_ref, cols_ref, g_ref, out_ref,
    rows_v, cols_v, idx_v, vals_v, g_v, o_v, acc_v, sum_v, tmp_v, red_v,
    all_sh, fin_sh, sem,
):
    c_idx = lax.axis_index("c")
    s_idx = lax.axis_index("s")
    base = s_idx * CHUNK
    gbase = c_idx * NNZ + base

    pltpu.sync_copy(rows_ref.at[pl.ds(base, CHUNK)], rows_v)
    pltpu.sync_copy(cols_ref.at[pl.ds(base, CHUNK)], cols_v)
    pltpu.sync_copy(g_ref.at[pl.ds(gbase, CHUNK)], g_v)

    def build(r, carry):
        for u in range(8):
            off = r * GCOLS + u * 16
            rv = rows_v[pl.ds(off, 16)]
            cv = cols_v[pl.ds(off, 16)]
            idx_v[r, pl.ds(u * 16, 16)] = (
                (cv >> 7) * (N_USERS * 128) + rv * 128 + (cv & 127)
            )
        return carry

    lax.fori_loop(0, GROWS, build, 0)

    def gather_from(tref):
        def wave(w, carry):
            for u in range(NSUB):
                j = w * NSUB + u
                pltpu.make_async_copy(
                    tref.at[idx_v.at[j]], vals_v.at[j], sem
                ).start()
            for u in range(NSUB):
                j = w * NSUB + u
                pltpu.make_async_copy(
                    tref.at[idx_v.at[j]], vals_v.at[j], sem
                ).wait()
            return carry

        lax.fori_loop(0, GROWS // NSUB, wave, 0)

    @pl.when(c_idx == 0)
    def _():
        gather_from(m1_ref)

    @pl.when(c_idx == 1)
    def _():
        gather_from(m2_ref)

    neg = jnp.full((16,), -3.0e38, jnp.float32)

    def init_acc(i, carry):
        acc_v[pl.ds(i * 16, 16)] = neg
        return carry

    lax.fori_loop(0, SEGS // 16, init_acc, 0)

    def seg_at(off):
        rv = rows_v[pl.ds(off, 16)]
        cv = cols_v[pl.ds(off, 16)]
        return rv + c_idx * (cv - rv)

    def p_max(r, carry):
        for u in range(8):
            off = r * GCOLS + u * 16
            seg = seg_at(off)
            w16 = vals_v[r, pl.ds(u * 16, 16)]
            g16 = g_v[pl.ds(off, 16)]
            logit = (w16 - g16) * INV_TAU
            vals_v[r, pl.ds(u * 16, 16)] = logit

            cur = plsc.load_gather(acc_v, [seg])
            plsc.store_scatter(acc_v, [seg], jnp.maximum(cur, logit))
            for _ in range(2):
                chk = plsc.load_gather(acc_v, [seg])
                plsc.store_scatter(acc_v, [seg], logit, mask=chk < logit)
        return carry

    lax.fori_loop(0, GROWS, p_max, 0)

    def combine(local_v, is_max):
        pltpu.sync_copy(local_v, all_sh.at[s_idx])
        plsc.subcore_barrier()
        sbase = s_idx * SLICE
        pltpu.sync_copy(all_sh.at[0, pl.ds(sbase, SLICE)], red_v)

        def fold(t, carry):
            pltpu.sync_copy(all_sh.at[t, pl.ds(sbase, SLICE)], tmp_v)
            for u in range(SLICE // 16):
                a = red_v[pl.ds(u * 16, 16)]
                b = tmp_v[pl.ds(u * 16, 16)]
                red_v[pl.ds(u * 16, 16)] = (
                    jnp.maximum(a, b) if is_max else a + b
                )
            return carry

        lax.fori_loop(1, NSUB, fold, 0)
        pltpu.sync_copy(red_v, fin_sh.at[pl.ds(sbase, SLICE)])
        plsc.subcore_barrier()
        pltpu.sync_copy(fin_sh, local_v)

    combine(acc_v, True)

    def init_sum(i, carry):
        sum_v[pl.ds(i * 16, 16)] = jnp.zeros((16,), jnp.float32)
        return carry

    lax.fori_loop(0, SEGS // 16, init_sum, 0)

    def p_exp(r, carry):
        for u in range(8):
            off = r * GCOLS + u * 16
            seg = seg_at(off)
            logit = vals_v[r, pl.ds(u * 16, 16)]
            m = plsc.load_gather(acc_v, [seg])
            e = jnp.exp(logit - m)
            vals_v[r, pl.ds(u * 16, 16)] = e
            plsc.addupdate_scatter(sum_v, [seg], e)
        return carry

    lax.fori_loop(0, GROWS, p_exp, 0)

    combine(sum_v, False)

    def p_out(r, carry):
        for u in range(8):
            off = r * GCOLS + u * 16
            seg = seg_at(off)
            e = vals_v[r, pl.ds(u * 16, 16)]
            ssum = plsc.load_gather(sum_v, [seg])
            o_v[pl.ds(off, 16)] = e / ssum
        return carry

    lax.fori_loop(0, GROWS, p_out, 0)
    pltpu.sync_copy(o_v, out_ref.at[pl.ds(gbase, CHUNK)])


def _sc_softmax(m1f, m2f, rows, cols, g):
    mesh = plsc.VectorSubcoreMesh(core_axis_name="c", subcore_axis_name="s")
    fn = functools.partial(
        pl.kernel,
        out_type=jax.ShapeDtypeStruct((2 * NNZ,), jnp.float32),
        mesh=mesh,
        compiler_params=pltpu.CompilerParams(needs_layout_passes=False),
        scratch_types=[
            pltpu.VMEM((CHUNK,), jnp.int32),
            pltpu.VMEM((CHUNK,), jnp.int32),
            pltpu.VMEM((GROWS, GCOLS), jnp.int32),
            pltpu.VMEM((GROWS, GCOLS), jnp.float32),
            pltpu.VMEM((CHUNK,), jnp.float32),
            pltpu.VMEM((CHUNK,), jnp.float32),
            pltpu.VMEM((SEGS,), jnp.float32),
            pltpu.VMEM((SEGS,), jnp.float32),
            pltpu.VMEM((SLICE,), jnp.float32),
            pltpu.VMEM((SLICE,), jnp.float32),
            pltpu.VMEM_SHARED((NSUB, SEGS), jnp.float32),
            pltpu.VMEM_SHARED((SEGS,), jnp.float32),
            pltpu.SemaphoreType.DMA,
        ],
    )(_sc_body)
    return fn(m1f, m2f, rows, cols, g)


def kernel(user_embed, item_embed, ui_rows, ui_cols, WQ, bQ, WK, bK):
    u1 = jax.random.uniform(
        jax.random.fold_in(jax.random.key(42), 1), (NNZ,),
        minval=1e-9, maxval=1.0 - 1e-9,
    )
    u2 = jax.random.uniform(
        jax.random.fold_in(jax.random.key(42), 2), (NNZ,),
        minval=1e-9, maxval=1.0 - 1e-9,
    )
    g = jnp.concatenate([jnp.log(-jnp.log(u1)), jnp.log(-jnp.log(u2))])

    X = jnp.concatenate([user_embed, item_embed], axis=0)
    QX, KX = _projections(X, WQ, bQ, WK, bK)
    M1, M2 = _dense_logits(QX, KX)
    return _sc_softmax(
        M1.reshape(-1), M2.reshape(-1), ui_rows, ui_cols, g
    )


# speedup vs baseline: 42.7762x; 1.0325x over previous
import functools

import jax
import jax.numpy as jnp
import numpy as np
from jax import lax
from jax.experimental import pallas as pl
from jax.experimental.pallas import tpu as pltpu
from jax.experimental.pallas import tpu_sc as plsc

N_USERS = 4096
N_ITEMS = 8192
NNZ = 262144
EMBED = 128
ATT = 128
TAU = 0.5
INV_TAU = 2.0

NSUB = 16
CHUNK = NNZ // NSUB
GROWS = 128
GCOLS = 128
SEGS = N_ITEMS
SLICE = SEGS // NSUB


def _proj_body(x_ref, wq_ref, bq_ref, wk_ref, bk_ref, q_ref, k_ref):
    x = x_ref[...]
    q_ref[...] = (
        jnp.dot(x, wq_ref[...], preferred_element_type=jnp.float32) + bq_ref[...]
    )
    k_ref[...] = (
        jnp.dot(x, wk_ref[...], preferred_element_type=jnp.float32) + bk_ref[...]
    )


def _projections(X, WQ, bQ, WK, bK):
    n = X.shape[0]
    blk = 1024
    grid = (n // blk,)
    return pl.pallas_call(
        _proj_body,
        grid=grid,
        in_specs=[
            pl.BlockSpec((blk, EMBED), lambda i: (i, 0)),
            pl.BlockSpec((EMBED, ATT), lambda i: (0, 0)),
            pl.BlockSpec((1, ATT), lambda i: (0, 0)),
            pl.BlockSpec((EMBED, ATT), lambda i: (0, 0)),
            pl.BlockSpec((1, ATT), lambda i: (0, 0)),
        ],
        out_specs=[
            pl.BlockSpec((blk, ATT), lambda i: (i, 0)),
            pl.BlockSpec((blk, ATT), lambda i: (i, 0)),
        ],
        out_shape=[
            jax.ShapeDtypeStruct((n, ATT), jnp.float32),
            jax.ShapeDtypeStruct((n, ATT), jnp.float32),
        ],
    )(X, WQ, bQ.reshape(1, ATT), WK, bK.reshape(1, ATT))


_GB = NNZ // (N_ITEMS // 1024)


def _proj_item_body(
    x_ref, wq_ref, bq_ref, wk_ref, bk_ref, u1_ref, u2_ref,
    q_ref, k_ref, g1_ref, g2_ref,
):
    x = x_ref[...]
    q_ref[...] = (
        jnp.dot(x, wq_ref[...], preferred_element_type=jnp.float32) + bq_ref[...]
    )
    k_ref[...] = (
        jnp.dot(x, wk_ref[...], preferred_element_type=jnp.float32) + bk_ref[...]
    )
    g1_ref[...] = jnp.log(-jnp.log(u1_ref[...]))
    g2_ref[...] = jnp.log(-jnp.log(u2_ref[...]))


def _proj_item_gumbel(X, WQ, bQ, WK, bK, u1, u2):
    n = X.shape[0]
    blk = 1024
    grid = (n // blk,)
    gr = _GB // 128
    u1 = u1.reshape(grid[0], gr, 128)
    u2 = u2.reshape(grid[0], gr, 128)
    return pl.pallas_call(
        _proj_item_body,
        grid=grid,
        in_specs=[
            pl.BlockSpec((blk, EMBED), lambda i: (i, 0)),
            pl.BlockSpec((EMBED, ATT), lambda i: (0, 0)),
            pl.BlockSpec((1, ATT), lambda i: (0, 0)),
            pl.BlockSpec((EMBED, ATT), lambda i: (0, 0)),
            pl.BlockSpec((1, ATT), lambda i: (0, 0)),
            pl.BlockSpec((1, gr, 128), lambda i: (i, 0, 0)),
            pl.BlockSpec((1, gr, 128), lambda i: (i, 0, 0)),
        ],
        out_specs=[
            pl.BlockSpec((blk, ATT), lambda i: (i, 0)),
            pl.BlockSpec((blk, ATT), lambda i: (i, 0)),
            pl.BlockSpec((1, gr, 128), lambda i: (i, 0, 0)),
            pl.BlockSpec((1, gr, 128), lambda i: (i, 0, 0)),
        ],
        out_shape=[
            jax.ShapeDtypeStruct((n, ATT), jnp.float32),
            jax.ShapeDtypeStruct((n, ATT), jnp.float32),
            jax.ShapeDtypeStruct((grid[0], gr, 128), jnp.float32),
            jax.ShapeDtypeStruct((grid[0], gr, 128), jnp.float32),
        ],
    )(X, WQ, bQ.reshape(1, ATT), WK, bK.reshape(1, ATT), u1, u2)


_SLAB = 128
_NSLAB = N_ITEMS // _SLAB


def _dense_body(qu_ref, ku_ref, qi_ref, ki_ref, m1_ref, m2_ref):
    dn = (((1,), (1,)), ((), ()))
    m1_ref[0] = lax.dot_general(
        qu_ref[...], ki_ref[...], dn, preferred_element_type=jnp.float32
    )
    m2_ref[0] = lax.dot_general(
        ku_ref[...], qi_ref[...], dn, preferred_element_type=jnp.float32
    )


def _dense_logits(Qu, Ku, Qi, Ki):
    grid = (_NSLAB,)
    return pl.pallas_call(
        _dense_body,
        grid=grid,
        in_specs=[
            pl.BlockSpec((N_USERS, ATT), lambda c: (0, 0)),
            pl.BlockSpec((N_USERS, ATT), lambda c: (0, 0)),
            pl.BlockSpec((_SLAB, ATT), lambda c: (c, 0)),
            pl.BlockSpec((_SLAB, ATT), lambda c: (c, 0)),
        ],
        out_specs=[
            pl.BlockSpec((1, N_USERS, _SLAB), lambda c: (c, 0, 0)),
            pl.BlockSpec((1, N_USERS, _SLAB), lambda c: (c, 0, 0)),
        ],
        out_shape=[
            jax.ShapeDtypeStruct((_NSLAB, N_USERS, _SLAB), jnp.float32),
            jax.ShapeDtypeStruct((_NSLAB, N_USERS, _SLAB), jnp.float32),
        ],
    )(Qu, Ku, Qi, Ki)


def _sc_body(
    m1_ref, m2_ref, rows_ref, cols_ref, g1_ref, g2_ref, out_ref,
    rows_v, cols_v, idx_v, vals_v, g_v, o_v, acc_v, sum_v, tmp_v, red_v,
    all_sh, fin_sh, sem,
):
    c_idx = lax.axis_index("c")
    s_idx = lax.axis_index("s")
    base = s_idx * CHUNK
    gbase = c_idx * NNZ + base

    pltpu.sync_copy(rows_ref.at[pl.ds(base, CHUNK)], rows_v)
    pltpu.sync_copy(cols_ref.at[pl.ds(base, CHUNK)], cols_v)

    @pl.when(c_idx == 0)
    def _():
        pltpu.sync_copy(g1_ref.at[pl.ds(base, CHUNK)], g_v)

    @pl.when(c_idx == 1)
    def _():
        pltpu.sync_copy(g2_ref.at[pl.ds(base, CHUNK)], g_v)

    NWAVE = GROWS // NSUB

    def gather_from(tref):
        def wave(w, carry):
            for u in range(NSUB):
                j = w * NSUB + u
                for k in range(8):
                    off = j * GCOLS + k * 16
                    rv = rows_v[pl.ds(off, 16)]
                    cv = cols_v[pl.ds(off, 16)]
                    idx_v[j, pl.ds(k * 16, 16)] = (
                        (cv >> 7) * (N_USERS * 128) + rv * 128 + (cv & 127)
                    )
                pltpu.make_async_copy(
                    tref.at[idx_v.at[j]], vals_v.at[j], sem
                ).start()

            @pl.when(w > 0)
            def _():
                for u in range(NSUB):
                    j2 = (w - 1) * NSUB + u
                    pltpu.make_async_copy(
                        tref.at[idx_v.at[j2]], vals_v.at[j2], sem
                    ).wait()

            return carry

        lax.fori_loop(0, NWAVE, wave, 0)

        neg = jnp.full((16,), -3.0e38, jnp.float32)
        zero = jnp.zeros((16,), jnp.float32)

        def init_accs(i, carry):
            for k in range(4):
                acc_v[pl.ds((i * 4 + k) * 16, 16)] = neg
                sum_v[pl.ds((i * 4 + k) * 16, 16)] = zero
            return carry

        lax.fori_loop(0, SEGS // 64, init_accs, 0)

        for u in range(NSUB):
            j2 = (NWAVE - 1) * NSUB + u
            pltpu.make_async_copy(
                tref.at[idx_v.at[j2]], vals_v.at[j2], sem
            ).wait()

    @pl.when(c_idx == 0)
    def _():
        gather_from(m1_ref)

    @pl.when(c_idx == 1)
    def _():
        gather_from(m2_ref)

    def p_max(r, carry):
        for u in range(8):
            off = r * GCOLS + u * 16
            rv = rows_v[pl.ds(off, 16)]
            cv = cols_v[pl.ds(off, 16)]
            seg = rv + c_idx * (cv - rv)
            idx_v[r, pl.ds(u * 16, 16)] = seg
            w16 = vals_v[r, pl.ds(u * 16, 16)]
            g16 = g_v[pl.ds(off, 16)]
            logit = (w16 - g16) * INV_TAU
            vals_v[r, pl.ds(u * 16, 16)] = logit

            cur = plsc.load_gather(acc_v, [seg])
            plsc.store_scatter(acc_v, [seg], jnp.maximum(cur, logit))
            for _ in range(2):
                chk = plsc.load_gather(acc_v, [seg])
                plsc.store_scatter(acc_v, [seg], logit, mask=chk < logit)
        return carry

    lax.fori_loop(0, GROWS, p_max, 0)

    def combine(local_v, is_max):
        pltpu.sync_copy(local_v, all_sh.at[s_idx])
        plsc.subcore_barrier()
        sbase = s_idx * SLICE
        pltpu.sync_copy(all_sh.at[0, pl.ds(sbase, SLICE)], red_v)

        def fold(t, carry):
            pltpu.sync_copy(all_sh.at[t, pl.ds(sbase, SLICE)], tmp_v)
            for u in range(SLICE // 16):
                a = red_v[pl.ds(u * 16, 16)]
                b = tmp_v[pl.ds(u * 16, 16)]
                red_v[pl.ds(u * 16, 16)] = (
                    jnp.maximum(a, b) if is_max else a + b
                )
            return carry

        lax.fori_loop(1, NSUB, fold, 0)
        pltpu.sync_copy(red_v, fin_sh.at[pl.ds(sbase, SLICE)])
        plsc.subcore_barrier()
        pltpu.sync_copy(fin_sh, local_v)

    combine(acc_v, True)

    def p_exp(r, carry):
        for u in range(8):
            seg = idx_v[r, pl.ds(u * 16, 16)]
            logit = vals_v[r, pl.ds(u * 16, 16)]
            m = plsc.load_gather(acc_v, [seg])
            e = jnp.exp(logit - m)
            vals_v[r, pl.ds(u * 16, 16)] = e
            plsc.addupdate_scatter(sum_v, [seg], e)
        return carry

    lax.fori_loop(0, GROWS, p_exp, 0)

    combine(sum_v, False)

    def p_out(r, carry):
        for u in range(8):
            off = r * GCOLS + u * 16
            seg = idx_v[r, pl.ds(u * 16, 16)]
            e = vals_v[r, pl.ds(u * 16, 16)]
            ssum = plsc.load_gather(sum_v, [seg])
            o_v[pl.ds(off, 16)] = e / ssum
        return carry

    lax.fori_loop(0, GROWS, p_out, 0)
    pltpu.sync_copy(o_v, out_ref.at[pl.ds(gbase, CHUNK)])


def _sc_softmax(m1f, m2f, rows, cols, g1, g2):
    mesh = plsc.VectorSubcoreMesh(core_axis_name="c", subcore_axis_name="s")
    fn = functools.partial(
        pl.kernel,
        out_type=jax.ShapeDtypeStruct((2 * NNZ,), jnp.float32),
        mesh=mesh,
        compiler_params=pltpu.CompilerParams(needs_layout_passes=False),
        scratch_types=[
            pltpu.VMEM((CHUNK,), jnp.int32),
            pltpu.VMEM((CHUNK,), jnp.int32),
            pltpu.VMEM((GROWS, GCOLS), jnp.int32),
            pltpu.VMEM((GROWS, GCOLS), jnp.float32),
            pltpu.VMEM((CHUNK,), jnp.float32),
            pltpu.VMEM((CHUNK,), jnp.float32),
            pltpu.VMEM((SEGS,), jnp.float32),
            pltpu.VMEM((SEGS,), jnp.float32),
            pltpu.VMEM((SLICE,), jnp.float32),
            pltpu.VMEM((SLICE,), jnp.float32),
            pltpu.VMEM_SHARED((NSUB, SEGS), jnp.float32),
            pltpu.VMEM_SHARED((SEGS,), jnp.float32),
            pltpu.SemaphoreType.DMA,
        ],
    )(_sc_body)
    return fn(m1f, m2f, rows, cols, g1, g2)


def kernel(user_embed, item_embed, ui_rows, ui_cols, WQ, bQ, WK, bK):
    u1 = jax.random.uniform(
        jax.random.fold_in(jax.random.key(42), 1), (NNZ,),
        minval=1e-9, maxval=1.0 - 1e-9,
    )
    u2 = jax.random.uniform(
        jax.random.fold_in(jax.random.key(42), 2), (NNZ,),
        minval=1e-9, maxval=1.0 - 1e-9,
    )
    Qu, Ku = _projections(user_embed, WQ, bQ, WK, bK)
    Qi, Ki, g1, g2 = _proj_item_gumbel(item_embed, WQ, bQ, WK, bK, u1, u2)
    M1, M2 = _dense_logits(Qu, Ku, Qi, Ki)
    return _sc_softmax(
        M1.reshape(-1), M2.reshape(-1), ui_rows, ui_cols,
        g1.reshape(-1), g2.reshape(-1),
    )
